# Initial kernel scaffold; baseline (speedup 1.0000x reference)
import functools

import jax
import jax.numpy as jnp
from jax import lax
from jax.experimental import pallas as pl
from jax.experimental.pallas import tpu as pltpu
from jax.experimental.pallas import tpu_sc as plsc

N = 10000
E = 320000
C = 128
NP = 10240
NW = 32
K = 128
NCHUNK = 80
EPT = K * NCHUNK
E_PAD = NW * EPT
ROWS_PT = NP // 16

_mesh = plsc.VectorSubcoreMesh(core_axis_name="c", subcore_axis_name="s")


@functools.partial(
    pl.kernel,
    out_type=jax.ShapeDtypeStruct((NW, NP), jnp.float32),
    mesh=_mesh,
    scratch_types=[
        pltpu.VMEM((NCHUNK, K), jnp.int32),
        pltpu.VMEM((NP,), jnp.float32),
    ],
)
def _sc_hist(dst_hbm, out_hbm, dst_v, hist_v):
    cid = lax.axis_index("c")
    sid = lax.axis_index("s")
    wid = sid * 2 + cid
    pltpu.sync_copy(dst_hbm.at[wid], dst_v)

    zero16 = jnp.zeros((16,), jnp.float32)

    def zbody(i, _):
        hist_v[pl.ds(i * 16, 16)] = zero16
        return 0

    lax.fori_loop(0, NP // 16, zbody, 0)

    ones16 = jnp.ones((16,), jnp.float32)

    def ebody(j, _):
        for cc in range(K // 16):
            idx = dst_v[j, pl.ds(cc * 16, 16)]
            plsc.addupdate_scatter(hist_v, [idx], ones16)
        return 0

    lax.fori_loop(0, NCHUNK, ebody, 0)
    pltpu.sync_copy(hist_v, out_hbm.at[wid])


@functools.partial(
    pl.kernel,
    out_type=jax.ShapeDtypeStruct((2, NP, C), jnp.float32),
    mesh=_mesh,
    scratch_types=[
        pltpu.VMEM((NCHUNK, K), jnp.int32),
        pltpu.VMEM((NCHUNK, K), jnp.int32),
        pltpu.VMEM((K, C), jnp.float32),
        pltpu.VMEM_SHARED((NP, C), jnp.float32),
        pltpu.SemaphoreType.DMA,
    ],
)
def _sc_agg(hs_hbm, src_hbm, dst_hbm, zeros_hbm, out_hbm,
            src_v, dst_v, rows_v, acc_sh, sem):
    cid = lax.axis_index("c")
    sid = lax.axis_index("s")
    wid = sid * 2 + cid
    base = sid * ROWS_PT
    pltpu.sync_copy(zeros_hbm, acc_sh.at[pl.ds(base, ROWS_PT)])
    pltpu.sync_copy(src_hbm.at[wid], src_v)
    pltpu.sync_copy(dst_hbm.at[wid], dst_v)
    plsc.subcore_barrier()

    def body(j, _):
        pltpu.async_copy(hs_hbm.at[src_v.at[j]], rows_v, sem).wait()
        pltpu.sync_copy(rows_v, acc_sh.at[dst_v.at[j]], add=True)
        return 0

    lax.fori_loop(0, NCHUNK, body, 0)
    plsc.subcore_barrier()
    pltpu.sync_copy(acc_sh.at[pl.ds(base, ROWS_PT)],
                    out_hbm.at[cid, pl.ds(base, ROWS_PT)])


def _tc_prep_body(x_ref, w_ref, hists_ref, hs_ref, dfull_ref):
    h = jnp.dot(x_ref[...], w_ref[...], preferred_element_type=jnp.float32)
    deg = jnp.sum(hists_ref[...], axis=0) + 1.0
    d = lax.rsqrt(deg)
    hs_ref[...] = h * d[:, None]
    dfull_ref[...] = jnp.broadcast_to(d[:, None], hs_ref.shape)


_BLK = 1024


def _tc_prep(x_pad, W, hists):
    return pl.pallas_call(
        _tc_prep_body,
        grid=(NP // _BLK,),
        in_specs=[
            pl.BlockSpec((_BLK, C), lambda i: (i, 0)),
            pl.BlockSpec((C, C), lambda i: (0, 0)),
            pl.BlockSpec((NW, _BLK), lambda i: (0, i)),
        ],
        out_specs=[
            pl.BlockSpec((_BLK, C), lambda i: (i, 0)),
            pl.BlockSpec((_BLK, C), lambda i: (i, 0)),
        ],
        out_shape=[
            jax.ShapeDtypeStruct((NP, C), jnp.float32),
            jax.ShapeDtypeStruct((NP, C), jnp.float32),
        ],
    )(x_pad, W, hists)


def _tc_final_body(a0_ref, a1_ref, hs_ref, df_ref, b_ref, pw_ref, o_ref):
    t = df_ref[...] * (a0_ref[...] + a1_ref[...] + hs_ref[...]) + b_ref[...]
    o_ref[...] = jnp.where(t >= 0, t, pw_ref[...] * t)


def _tc_final(a0, a1, hs, dfull, b2, pw2):
    blk = pl.BlockSpec((_BLK, C), lambda i: (i, 0))
    vec = pl.BlockSpec((1, C), lambda i: (0, 0))
    return pl.pallas_call(
        _tc_final_body,
        grid=(NP // _BLK,),
        in_specs=[blk, blk, blk, blk, vec, vec],
        out_specs=blk,
        out_shape=jax.ShapeDtypeStruct((NP, C), jnp.float32),
    )(a0, a1, hs, dfull, b2, pw2)


def kernel(x, edge_index, W, b, prelu_w):
    ei = edge_index.astype(jnp.int32)
    pad = E_PAD - E
    src_p = jnp.concatenate(
        [ei[0], jnp.zeros((pad,), jnp.int32)]).reshape(NW, NCHUNK, K)
    dst_p = jnp.concatenate(
        [ei[1], jnp.full((pad,), N, jnp.int32)]).reshape(NW, NCHUNK, K)
    x_pad = jnp.pad(x, ((0, NP - N), (0, 0)))
    zeros_blk = jnp.zeros((ROWS_PT, C), jnp.float32)

    hists = _sc_hist(dst_p)
    hs, dfull = _tc_prep(x_pad, W, hists)
    acc = _sc_agg(hs, src_p, dst_p, zeros_blk)
    out = _tc_final(acc[0], acc[1], hs, dfull,
                    b.reshape(1, C), prelu_w.reshape(1, C))
    return out[:N]


# speedup vs baseline: 11.6959x; 11.6959x over previous
import functools

import jax
import jax.numpy as jnp
from jax import lax
from jax.experimental import pallas as pl
from jax.experimental.pallas import tpu as pltpu
from jax.experimental.pallas import tpu_sc as plsc

N = 10000
E = 320000
C = 128
NP = 10240
NW = 32
K = 128
NCHUNK = 80
EPT = K * NCHUNK
E_PAD = NW * EPT
ROWS_PT = NP // 16

_mesh = plsc.VectorSubcoreMesh(core_axis_name="c", subcore_axis_name="s")


@functools.partial(
    pl.kernel,
    out_type=jax.ShapeDtypeStruct((NW, NP), jnp.float32),
    mesh=_mesh,
    scratch_types=[
        pltpu.VMEM((NCHUNK, K), jnp.int32),
        pltpu.VMEM((NP,), jnp.float32),
    ],
    compiler_params=pltpu.CompilerParams(needs_layout_passes=False),
)
def _sc_hist(dst_hbm, out_hbm, dst_v, hist_v):
    cid = lax.axis_index("c")
    sid = lax.axis_index("s")
    wid = sid * 2 + cid
    pltpu.sync_copy(dst_hbm.at[wid], dst_v)

    zero16 = jnp.zeros((16,), jnp.float32)

    def zbody(i, _):
        hist_v[pl.ds(i * 16, 16)] = zero16
        return 0

    lax.fori_loop(0, NP // 16, zbody, 0)

    ones16 = jnp.ones((16,), jnp.float32)

    def ebody(j, _):
        for cc in range(K // 16):
            idx = dst_v[j, pl.ds(cc * 16, 16)]
            plsc.addupdate_scatter(hist_v, [idx], ones16)
        return 0

    lax.fori_loop(0, NCHUNK, ebody, 0)
    pltpu.sync_copy(hist_v, out_hbm.at[wid])


@functools.partial(
    pl.kernel,
    out_type=jax.ShapeDtypeStruct((2, NP, C), jnp.float32),
    mesh=_mesh,
    scratch_types=[
        pltpu.VMEM((NCHUNK, K), jnp.int32),
        pltpu.VMEM((NCHUNK, K), jnp.int32),
        pltpu.VMEM((K, C), jnp.float32),
        pltpu.VMEM_SHARED((NP, C), jnp.float32),
        pltpu.SemaphoreType.DMA,
    ],
)
def _sc_agg(hs_hbm, src_hbm, dst_hbm, zeros_hbm, out_hbm,
            src_v, dst_v, rows_v, acc_sh, sem):
    cid = lax.axis_index("c")
    sid = lax.axis_index("s")
    wid = sid * 2 + cid
    base = sid * ROWS_PT
    pltpu.sync_copy(zeros_hbm, acc_sh.at[pl.ds(base, ROWS_PT)])
    pltpu.sync_copy(src_hbm.at[wid], src_v)
    pltpu.sync_copy(dst_hbm.at[wid], dst_v)
    plsc.subcore_barrier()

    def body(j, _):
        pltpu.async_copy(hs_hbm.at[src_v.at[j]], rows_v, sem).wait()
        pltpu.sync_copy(rows_v, acc_sh.at[dst_v.at[j]], add=True)
        return 0

    lax.fori_loop(0, NCHUNK, body, 0)
    plsc.subcore_barrier()
    pltpu.sync_copy(acc_sh.at[pl.ds(base, ROWS_PT)],
                    out_hbm.at[cid, pl.ds(base, ROWS_PT)])


def _tc_prep_body(x_ref, w_ref, hists_ref, hs_ref, dfull_ref):
    h = jnp.dot(x_ref[...], w_ref[...], preferred_element_type=jnp.float32)
    deg = jnp.sum(hists_ref[...], axis=0) + 1.0
    d = lax.rsqrt(deg)
    hs_ref[...] = h * d[:, None]
    dfull_ref[...] = jnp.broadcast_to(d[:, None], hs_ref.shape)


_BLK = 1024


def _tc_prep(x_pad, W, hists):
    return pl.pallas_call(
        _tc_prep_body,
        grid=(NP // _BLK,),
        in_specs=[
            pl.BlockSpec((_BLK, C), lambda i: (i, 0)),
            pl.BlockSpec((C, C), lambda i: (0, 0)),
            pl.BlockSpec((NW, _BLK), lambda i: (0, i)),
        ],
        out_specs=[
            pl.BlockSpec((_BLK, C), lambda i: (i, 0)),
            pl.BlockSpec((_BLK, C), lambda i: (i, 0)),
        ],
        out_shape=[
            jax.ShapeDtypeStruct((NP, C), jnp.float32),
            jax.ShapeDtypeStruct((NP, C), jnp.float32),
        ],
    )(x_pad, W, hists)


def _tc_final_body(a0_ref, a1_ref, hs_ref, df_ref, b_ref, pw_ref, o_ref):
    t = df_ref[...] * (a0_ref[...] + a1_ref[...] + hs_ref[...]) + b_ref[...]
    o_ref[...] = jnp.where(t >= 0, t, pw_ref[...] * t)


def _tc_final(a0, a1, hs, dfull, b2, pw2):
    blk = pl.BlockSpec((_BLK, C), lambda i: (i, 0))
    vec = pl.BlockSpec((1, C), lambda i: (0, 0))
    return pl.pallas_call(
        _tc_final_body,
        grid=(NP // _BLK,),
        in_specs=[blk, blk, blk, blk, vec, vec],
        out_specs=blk,
        out_shape=jax.ShapeDtypeStruct((NP, C), jnp.float32),
    )(a0, a1, hs, dfull, b2, pw2)


def kernel(x, edge_index, W, b, prelu_w):
    ei = edge_index.astype(jnp.int32)
    pad = E_PAD - E
    src_p = jnp.concatenate(
        [ei[0], jnp.zeros((pad,), jnp.int32)]).reshape(NW, NCHUNK, K)
    dst_p = jnp.concatenate(
        [ei[1], jnp.full((pad,), N, jnp.int32)]).reshape(NW, NCHUNK, K)
    x_pad = jnp.pad(x, ((0, NP - N), (0, 0)))
    zeros_blk = jnp.zeros((ROWS_PT, C), jnp.float32)

    hists = _sc_hist(dst_p)
    hs, dfull = _tc_prep(x_pad, W, hists)
    acc = _sc_agg(hs, src_p, dst_p, zeros_blk)
    out = _tc_final(acc[0], acc[1], hs, dfull,
                    b.reshape(1, C), prelu_w.reshape(1, C))
    return out[:N]


# speedup vs baseline: 12.5003x; 1.0688x over previous
import functools

import jax
import jax.numpy as jnp
from jax import lax
from jax.experimental import pallas as pl
from jax.experimental.pallas import tpu as pltpu
from jax.experimental.pallas import tpu_sc as plsc

N = 10000
E = 320000
C = 128
NP = 10240
NW = 32
K = 128
NCHUNK = 80
EPT = K * NCHUNK
E_PAD = NW * EPT
ROWS_PT = NP // 16

_mesh = plsc.VectorSubcoreMesh(core_axis_name="c", subcore_axis_name="s")


@functools.partial(
    pl.kernel,
    out_type=jax.ShapeDtypeStruct((NW, NP), jnp.float32),
    mesh=_mesh,
    scratch_types=[
        pltpu.VMEM((NCHUNK, K), jnp.int32),
        pltpu.VMEM((NP,), jnp.float32),
    ],
    compiler_params=pltpu.CompilerParams(needs_layout_passes=False),
)
def _sc_hist(dst_hbm, out_hbm, dst_v, hist_v):
    cid = lax.axis_index("c")
    sid = lax.axis_index("s")
    wid = sid * 2 + cid
    pltpu.sync_copy(dst_hbm.at[wid], dst_v)

    zero16 = jnp.zeros((16,), jnp.float32)

    def zbody(i, _):
        hist_v[pl.ds(i * 16, 16)] = zero16
        return 0

    lax.fori_loop(0, NP // 16, zbody, 0)

    ones16 = jnp.ones((16,), jnp.float32)

    def ebody(j, _):
        for cc in range(K // 16):
            idx = dst_v[j, pl.ds(cc * 16, 16)]
            plsc.addupdate_scatter(hist_v, [idx], ones16)
        return 0

    lax.fori_loop(0, NCHUNK, ebody, 0)
    pltpu.sync_copy(hist_v, out_hbm.at[wid])


@functools.partial(
    pl.kernel,
    out_type=jax.ShapeDtypeStruct((2, NP, C), jnp.float32),
    mesh=_mesh,
    scratch_types=[
        pltpu.VMEM((NCHUNK // 2, K), jnp.int32),
        pltpu.VMEM((NCHUNK // 2, K), jnp.int32),
        pltpu.VMEM((2, K, C), jnp.float32),
        pltpu.VMEM_SHARED((NP, C), jnp.float32),
        pltpu.SemaphoreType.DMA,
        pltpu.SemaphoreType.DMA,
    ],
)
def _sc_agg(hs_hbm, src_hbm, dst_hbm, zeros_hbm, out_hbm,
            src_v, dst_v, rows_v, acc_sh, gsem, ssem):
    cid = lax.axis_index("c")
    sid = lax.axis_index("s")
    wid = sid * 2 + cid
    base = sid * ROWS_PT
    NC2 = NCHUNK // 2
    pltpu.sync_copy(zeros_hbm, acc_sh.at[pl.ds(base, ROWS_PT)])
    plsc.subcore_barrier()

    def start_g(j, b):
        pltpu.async_copy(hs_hbm.at[src_v.at[j]], rows_v.at[b], gsem)

    def wait_g(j, b):
        pltpu.make_async_copy(hs_hbm.at[src_v.at[j]], rows_v.at[b], gsem).wait()

    def start_s(j, b):
        pltpu.async_copy(rows_v.at[b], acc_sh.at[dst_v.at[j]], ssem, add=True)

    def wait_s(j, b):
        pltpu.make_async_copy(rows_v.at[b], acc_sh.at[dst_v.at[j]], ssem).wait()

    def run_phase():
        start_g(0, 0)
        wait_g(0, 0)
        start_s(0, 0)
        start_g(1, 1)

        def body(t, _):
            for i in range(2):
                j = 2 * t + 1 + i
                b = (1 + i) % 2
                wait_g(j, b)
                start_s(j, b)
                wait_s(j - 1, (b + 1) % 2)
                start_g(j + 1, (b + 1) % 2)
            return 0

        lax.fori_loop(0, (NC2 - 2) // 2, body, 0)
        wait_g(NC2 - 1, 1)
        start_s(NC2 - 1, 1)
        wait_s(NC2 - 2, 0)
        wait_s(NC2 - 1, 1)

    for p in range(2):
        pltpu.sync_copy(src_hbm.at[wid, pl.ds(p * NC2, NC2)], src_v)
        pltpu.sync_copy(dst_hbm.at[wid, pl.ds(p * NC2, NC2)], dst_v)
        run_phase()
    plsc.subcore_barrier()
    pltpu.sync_copy(acc_sh.at[pl.ds(base, ROWS_PT)],
                    out_hbm.at[cid, pl.ds(base, ROWS_PT)])


def _tc_prep_body(x_ref, w_ref, hists_ref, hs_ref, dfull_ref):
    h = jnp.dot(x_ref[...], w_ref[...], preferred_element_type=jnp.float32)
    deg = jnp.sum(hists_ref[...], axis=0) + 1.0
    d = lax.rsqrt(deg)
    hs_ref[...] = h * d[:, None]
    dfull_ref[...] = jnp.broadcast_to(d[:, None], hs_ref.shape)


_BLK = 1024


def _tc_prep(x_pad, W, hists):
    return pl.pallas_call(
        _tc_prep_body,
        grid=(NP // _BLK,),
        in_specs=[
            pl.BlockSpec((_BLK, C), lambda i: (i, 0)),
            pl.BlockSpec((C, C), lambda i: (0, 0)),
            pl.BlockSpec((NW, _BLK), lambda i: (0, i)),
        ],
        out_specs=[
            pl.BlockSpec((_BLK, C), lambda i: (i, 0)),
            pl.BlockSpec((_BLK, C), lambda i: (i, 0)),
        ],
        out_shape=[
            jax.ShapeDtypeStruct((NP, C), jnp.float32),
            jax.ShapeDtypeStruct((NP, C), jnp.float32),
        ],
    )(x_pad, W, hists)


def _tc_final_body(a0_ref, a1_ref, hs_ref, df_ref, b_ref, pw_ref, o_ref):
    t = df_ref[...] * (a0_ref[...] + a1_ref[...] + hs_ref[...]) + b_ref[...]
    o_ref[...] = jnp.where(t >= 0, t, pw_ref[...] * t)


def _tc_final(a0, a1, hs, dfull, b2, pw2):
    blk = pl.BlockSpec((_BLK, C), lambda i: (i, 0))
    vec = pl.BlockSpec((1, C), lambda i: (0, 0))
    return pl.pallas_call(
        _tc_final_body,
        grid=(NP // _BLK,),
        in_specs=[blk, blk, blk, blk, vec, vec],
        out_specs=blk,
        out_shape=jax.ShapeDtypeStruct((NP, C), jnp.float32),
    )(a0, a1, hs, dfull, b2, pw2)


def kernel(x, edge_index, W, b, prelu_w):
    ei = edge_index.astype(jnp.int32)
    pad = E_PAD - E
    src_p = jnp.concatenate(
        [ei[0], jnp.zeros((pad,), jnp.int32)]).reshape(NW, NCHUNK, K)
    dst_p = jnp.concatenate(
        [ei[1], jnp.full((pad,), N, jnp.int32)]).reshape(NW, NCHUNK, K)
    x_pad = jnp.pad(x, ((0, NP - N), (0, 0)))
    zeros_blk = jnp.zeros((ROWS_PT, C), jnp.float32)

    hists = _sc_hist(dst_p)
    hs, dfull = _tc_prep(x_pad, W, hists)
    acc = _sc_agg(hs, src_p, dst_p, zeros_blk)
    out = _tc_final(acc[0], acc[1], hs, dfull,
                    b.reshape(1, C), prelu_w.reshape(1, C))
    return out[:N]


# speedup vs baseline: 15.7633x; 1.2610x over previous
import functools

import jax
import jax.numpy as jnp
from jax import lax
from jax.experimental import pallas as pl
from jax.experimental.pallas import tpu as pltpu
from jax.experimental.pallas import tpu_sc as plsc

N = 10000
E = 320000
C = 128
NP = 10240
NW = 32
K = 128
NCHUNK = 80
TOT_CHUNKS = NW * NCHUNK
E_PAD = TOT_CHUNKS * K
ROWS_PT = NP // 16
CH_C0 = 120
CH_C1 = 40
assert 16 * (CH_C0 + CH_C1) == TOT_CHUNKS

_mesh = plsc.VectorSubcoreMesh(core_axis_name="c", subcore_axis_name="s")


@functools.partial(
    pl.kernel,
    out_type=jax.ShapeDtypeStruct((NW, NP), jnp.float32),
    mesh=_mesh,
    scratch_types=[
        pltpu.VMEM((NCHUNK, K), jnp.int32),
        pltpu.VMEM((NP,), jnp.float32),
    ],
    compiler_params=pltpu.CompilerParams(needs_layout_passes=False),
)
def _sc_hist(dst_hbm, out_hbm, dst_v, hist_v):
    cid = lax.axis_index("c")
    sid = lax.axis_index("s")
    wid = sid * 2 + cid
    pltpu.sync_copy(dst_hbm.at[pl.ds(wid * NCHUNK, NCHUNK)], dst_v)

    zero16 = jnp.zeros((16,), jnp.float32)

    def zbody(i, _):
        hist_v[pl.ds(i * 16, 16)] = zero16
        return 0

    lax.fori_loop(0, NP // 16, zbody, 0)

    ones16 = jnp.ones((16,), jnp.float32)

    def ebody(j, _):
        for cc in range(K // 16):
            idx = dst_v[j, pl.ds(cc * 16, 16)]
            plsc.addupdate_scatter(hist_v, [idx], ones16)
        return 0

    lax.fori_loop(0, NCHUNK, ebody, 0)
    pltpu.sync_copy(hist_v, out_hbm.at[wid])


@functools.partial(
    pl.kernel,
    out_type=jax.ShapeDtypeStruct((2, NP, C), jnp.float32),
    mesh=_mesh,
    scratch_types=[
        pltpu.VMEM((NCHUNK // 2, K), jnp.int32),
        pltpu.VMEM((NCHUNK // 2, K), jnp.int32),
        pltpu.VMEM((2, K, C), jnp.float32),
        pltpu.VMEM_SHARED((NP, C), jnp.float32),
        pltpu.SemaphoreType.DMA,
        pltpu.SemaphoreType.DMA,
    ],
)
def _sc_agg(hs_hbm, src_hbm, dst_hbm, zeros_hbm, out_hbm,
            src_v, dst_v, rows_v, acc_sh, gsem, ssem):
    cid = lax.axis_index("c")
    sid = lax.axis_index("s")
    base = sid * ROWS_PT
    pltpu.sync_copy(zeros_hbm, acc_sh.at[pl.ds(base, ROWS_PT)])
    plsc.subcore_barrier()

    def start_g(j, b):
        pltpu.async_copy(hs_hbm.at[src_v.at[j]], rows_v.at[b], gsem)

    def wait_g(j, b):
        pltpu.make_async_copy(hs_hbm.at[src_v.at[j]], rows_v.at[b], gsem).wait()

    def start_s(j, b):
        pltpu.async_copy(rows_v.at[b], acc_sh.at[dst_v.at[j]], ssem, add=True)

    def wait_s(j, b):
        pltpu.make_async_copy(rows_v.at[b], acc_sh.at[dst_v.at[j]], ssem).wait()

    def run_phase(start, n):
        pltpu.sync_copy(src_hbm.at[pl.ds(start, n)], src_v.at[pl.ds(0, n)])
        pltpu.sync_copy(dst_hbm.at[pl.ds(start, n)], dst_v.at[pl.ds(0, n)])
        start_g(0, 0)
        wait_g(0, 0)
        start_s(0, 0)
        start_g(1, 1)

        def body(t, _):
            for i in range(2):
                j = 2 * t + 1 + i
                b = (1 + i) % 2
                wait_g(j, b)
                start_s(j, b)
                wait_s(j - 1, (b + 1) % 2)
                start_g(j + 1, (b + 1) % 2)
            return 0

        lax.fori_loop(0, (n - 2) // 2, body, 0)
        wait_g(n - 1, 1)
        start_s(n - 1, 1)
        wait_s(n - 2, 0)
        wait_s(n - 1, 1)

    @pl.when(cid == 0)
    def _():
        s0 = sid * CH_C0
        for off in range(0, CH_C0, 40):
            run_phase(s0 + off, min(40, CH_C0 - off))

    @pl.when(cid == 1)
    def _():
        s0 = 16 * CH_C0 + sid * CH_C1
        for off in range(0, CH_C1, 40):
            run_phase(s0 + off, min(40, CH_C1 - off))

    plsc.subcore_barrier()
    pltpu.sync_copy(acc_sh.at[pl.ds(base, ROWS_PT)],
                    out_hbm.at[cid, pl.ds(base, ROWS_PT)])


def _tc_prep_body(x_ref, w_ref, hists_ref, hs_ref, dfull_ref):
    h = jnp.dot(x_ref[...], w_ref[...], preferred_element_type=jnp.float32)
    deg = jnp.sum(hists_ref[...], axis=0) + 1.0
    d = lax.rsqrt(deg)
    hs_ref[...] = h * d[:, None]
    dfull_ref[...] = jnp.broadcast_to(d[:, None], hs_ref.shape)


_BLK = 1024


def _tc_prep(x_pad, W, hists):
    return pl.pallas_call(
        _tc_prep_body,
        grid=(NP // _BLK,),
        in_specs=[
            pl.BlockSpec((_BLK, C), lambda i: (i, 0)),
            pl.BlockSpec((C, C), lambda i: (0, 0)),
            pl.BlockSpec((NW, _BLK), lambda i: (0, i)),
        ],
        out_specs=[
            pl.BlockSpec((_BLK, C), lambda i: (i, 0)),
            pl.BlockSpec((_BLK, C), lambda i: (i, 0)),
        ],
        out_shape=[
            jax.ShapeDtypeStruct((NP, C), jnp.float32),
            jax.ShapeDtypeStruct((NP, C), jnp.float32),
        ],
    )(x_pad, W, hists)


def _tc_final_body(a0_ref, a1_ref, hs_ref, df_ref, b_ref, pw_ref, o_ref):
    t = df_ref[...] * (a0_ref[...] + a1_ref[...] + hs_ref[...]) + b_ref[...]
    o_ref[...] = jnp.where(t >= 0, t, pw_ref[...] * t)


def _tc_final(a0, a1, hs, dfull, b2, pw2):
    blk = pl.BlockSpec((_BLK, C), lambda i: (i, 0))
    vec = pl.BlockSpec((1, C), lambda i: (0, 0))
    return pl.pallas_call(
        _tc_final_body,
        grid=(NP // _BLK,),
        in_specs=[blk, blk, blk, blk, vec, vec],
        out_specs=blk,
        out_shape=jax.ShapeDtypeStruct((NP, C), jnp.float32),
    )(a0, a1, hs, dfull, b2, pw2)


def kernel(x, edge_index, W, b, prelu_w):
    ei = edge_index.astype(jnp.int32)
    pad = E_PAD - E
    src_p = jnp.concatenate(
        [ei[0], jnp.zeros((pad,), jnp.int32)]).reshape(TOT_CHUNKS, K)
    dst_p = jnp.concatenate(
        [ei[1], jnp.full((pad,), N, jnp.int32)]).reshape(TOT_CHUNKS, K)
    x_pad = jnp.pad(x, ((0, NP - N), (0, 0)))
    zeros_blk = jnp.zeros((ROWS_PT, C), jnp.float32)

    hists = _sc_hist(dst_p)
    hs, dfull = _tc_prep(x_pad, W, hists)
    acc = _sc_agg(hs, src_p, dst_p, zeros_blk)
    out = _tc_final(acc[0], acc[1], hs, dfull,
                    b.reshape(1, C), prelu_w.reshape(1, C))
    return out[:N]


# speedup vs baseline: 36.9163x; 2.3419x over previous
import functools

import jax
import jax.numpy as jnp
from jax import lax
from jax.experimental import pallas as pl
from jax.experimental.pallas import tpu as pltpu
from jax.experimental.pallas import tpu_sc as plsc

N = 10000
E = 320000
C = 128
NP = 10240
NW = 32
K = 128
NCHUNK = 80
TOT_CHUNKS = NW * NCHUNK
E_PAD = TOT_CHUNKS * K
ROWS_PT = NP // 16
CH_C0 = 80
CH_C1 = 80
assert 16 * (CH_C0 + CH_C1) == TOT_CHUNKS

_mesh = plsc.VectorSubcoreMesh(core_axis_name="c", subcore_axis_name="s")


@functools.partial(
    pl.kernel,
    out_type=jax.ShapeDtypeStruct((NW, NP), jnp.float32),
    mesh=_mesh,
    scratch_types=[
        pltpu.VMEM((NCHUNK, K), jnp.int32),
        pltpu.VMEM((NP,), jnp.float32),
    ],
    compiler_params=pltpu.CompilerParams(needs_layout_passes=False),
)
def _sc_hist(dst_hbm, out_hbm, dst_v, hist_v):
    cid = lax.axis_index("c")
    sid = lax.axis_index("s")
    wid = sid * 2 + cid
    pltpu.sync_copy(dst_hbm.at[pl.ds(wid * NCHUNK, NCHUNK)], dst_v)

    zero16 = jnp.zeros((16,), jnp.float32)

    def zbody(i, _):
        hist_v[pl.ds(i * 16, 16)] = zero16
        return 0

    lax.fori_loop(0, NP // 16, zbody, 0)

    ones16 = jnp.ones((16,), jnp.float32)

    def ebody(j, _):
        for cc in range(K // 16):
            idx = dst_v[j, pl.ds(cc * 16, 16)]
            plsc.addupdate_scatter(hist_v, [idx], ones16)
        return 0

    lax.fori_loop(0, NCHUNK, ebody, 0)
    pltpu.sync_copy(hist_v, out_hbm.at[wid])


@functools.partial(
    pl.kernel,
    out_type=jax.ShapeDtypeStruct((2, NP, C), jnp.float32),
    mesh=_mesh,
    scratch_types=[
        pltpu.VMEM((NCHUNK // 2, K), jnp.int32),
        pltpu.VMEM((NCHUNK // 2, K), jnp.int32),
        pltpu.VMEM((2, K, C), jnp.float32),
        pltpu.VMEM_SHARED((NP, C), jnp.float32),
        pltpu.SemaphoreType.DMA,
        pltpu.SemaphoreType.DMA,
    ],
)
def _sc_agg(hs_hbm, src_hbm, dst_hbm, zeros_hbm, out_hbm,
            src_v, dst_v, rows_v, acc_sh, gsem, ssem):
    cid = lax.axis_index("c")
    sid = lax.axis_index("s")
    base = sid * ROWS_PT
    pltpu.sync_copy(zeros_hbm, acc_sh.at[pl.ds(base, ROWS_PT)])
    plsc.subcore_barrier()

    def start_g(j, b):
        pltpu.async_copy(hs_hbm.at[src_v.at[j]], rows_v.at[b], gsem)

    def wait_g(j, b):
        pltpu.make_async_copy(hs_hbm.at[src_v.at[j]], rows_v.at[b], gsem).wait()

    def start_s(j, b):
        pltpu.async_copy(rows_v.at[b], acc_sh.at[dst_v.at[j]], ssem, add=True)

    def wait_s(j, b):
        pltpu.make_async_copy(rows_v.at[b], acc_sh.at[dst_v.at[j]], ssem).wait()

    def run_phase(start, n):
        pltpu.sync_copy(src_hbm.at[pl.ds(start, n)], src_v.at[pl.ds(0, n)])
        pltpu.sync_copy(dst_hbm.at[pl.ds(start, n)], dst_v.at[pl.ds(0, n)])
        start_g(0, 0)
        wait_g(0, 0)
        start_s(0, 0)
        start_g(1, 1)

        def body(t, _):
            for i in range(2):
                j = 2 * t + 1 + i
                b = (1 + i) % 2
                wait_g(j, b)
                start_s(j, b)
                wait_s(j - 1, (b + 1) % 2)
                start_g(j + 1, (b + 1) % 2)
            return 0

        lax.fori_loop(0, (n - 2) // 2, body, 0)
        wait_g(n - 1, 1)
        start_s(n - 1, 1)
        wait_s(n - 2, 0)
        wait_s(n - 1, 1)

    @pl.when(cid == 0)
    def _():
        s0 = sid * CH_C0
        for off in range(0, CH_C0, 40):
            run_phase(s0 + off, min(40, CH_C0 - off))

    @pl.when(cid == 1)
    def _():
        s0 = 16 * CH_C0 + sid * CH_C1
        for off in range(0, CH_C1, 40):
            run_phase(s0 + off, min(40, CH_C1 - off))

    plsc.subcore_barrier()
    pltpu.sync_copy(acc_sh.at[pl.ds(base, ROWS_PT)],
                    out_hbm.at[cid, pl.ds(base, ROWS_PT)])


def _tc_prep_body(x_ref, w_ref, hists_ref, hs_ref, dfull_ref):
    h = jnp.dot(x_ref[...], w_ref[...], preferred_element_type=jnp.float32)
    deg = jnp.sum(hists_ref[...], axis=0) + 1.0
    d = lax.rsqrt(deg)
    hs_ref[...] = h * d[:, None]
    dfull_ref[...] = jnp.broadcast_to(d[:, None], hs_ref.shape)


_BLK = 1024


def _tc_prep(x_pad, W, hists):
    return pl.pallas_call(
        _tc_prep_body,
        grid=(NP // _BLK,),
        in_specs=[
            pl.BlockSpec((_BLK, C), lambda i: (i, 0)),
            pl.BlockSpec((C, C), lambda i: (0, 0)),
            pl.BlockSpec((NW, _BLK), lambda i: (0, i)),
        ],
        out_specs=[
            pl.BlockSpec((_BLK, C), lambda i: (i, 0)),
            pl.BlockSpec((_BLK, C), lambda i: (i, 0)),
        ],
        out_shape=[
            jax.ShapeDtypeStruct((NP, C), jnp.float32),
            jax.ShapeDtypeStruct((NP, C), jnp.float32),
        ],
    )(x_pad, W, hists)


def _tc_final_body(a0_ref, a1_ref, hs_ref, df_ref, b_ref, pw_ref, o_ref):
    t = df_ref[...] * (a0_ref[...] + a1_ref[...] + hs_ref[...]) + b_ref[...]
    o_ref[...] = jnp.where(t >= 0, t, pw_ref[...] * t)


def _tc_final(a0, a1, hs, dfull, b2, pw2):
    blk = pl.BlockSpec((_BLK, C), lambda i: (i, 0))
    vec = pl.BlockSpec((1, C), lambda i: (0, 0))
    return pl.pallas_call(
        _tc_final_body,
        grid=(NP // _BLK,),
        in_specs=[blk, blk, blk, blk, vec, vec],
        out_specs=blk,
        out_shape=jax.ShapeDtypeStruct((NP, C), jnp.float32),
    )(a0, a1, hs, dfull, b2, pw2)


def kernel(x, edge_index, W, b, prelu_w):
    ei = edge_index.astype(jnp.int32)
    pad = E_PAD - E
    pad_iota = jnp.arange(pad, dtype=jnp.int32)
    src_p = jnp.concatenate(
        [ei[0], pad_iota % N]).reshape(TOT_CHUNKS, K)
    dst_p = jnp.concatenate(
        [ei[1], N + pad_iota % (NP - N)]).reshape(TOT_CHUNKS, K)
    x_pad = jnp.pad(x, ((0, NP - N), (0, 0)))
    zeros_blk = jnp.zeros((ROWS_PT, C), jnp.float32)

    hists = _sc_hist(dst_p)
    hs, dfull = _tc_prep(x_pad, W, hists)
    acc = _sc_agg(hs, src_p, dst_p, zeros_blk)
    out = _tc_final(acc[0], acc[1], hs, dfull,
                    b.reshape(1, C), prelu_w.reshape(1, C))
    return out[:N]


# speedup vs baseline: 41.7736x; 1.1316x over previous
import functools

import jax
import jax.numpy as jnp
from jax import lax
from jax.experimental import pallas as pl
from jax.experimental.pallas import tpu as pltpu
from jax.experimental.pallas import tpu_sc as plsc

N = 10000
E = 320000
C = 128
NP = 10240
NW = 32
K = 128
RCHUNK = E // K
NCHUNK = 80
PADC = NW * NCHUNK - RCHUNK
R31 = RCHUNK - 31 * NCHUNK
ROWS_PT = NP // 16
BLK = 1000

_mesh = plsc.VectorSubcoreMesh(core_axis_name="c", subcore_axis_name="s")


@functools.partial(
    pl.kernel,
    out_type=jax.ShapeDtypeStruct((NW, NP), jnp.float32),
    mesh=_mesh,
    scratch_types=[
        pltpu.VMEM((NCHUNK, K), jnp.int32),
        pltpu.VMEM((NP,), jnp.float32),
    ],
    compiler_params=pltpu.CompilerParams(needs_layout_passes=False),
)
def _sc_hist(ei_hbm, pad_dst_hbm, out_hbm, dst_v, hist_v):
    cid = lax.axis_index("c")
    sid = lax.axis_index("s")
    wid = sid * 2 + cid

    zero16 = jnp.zeros((16,), jnp.float32)

    def zbody(i, _):
        hist_v[pl.ds(i * 16, 16)] = zero16
        return 0

    lax.fori_loop(0, NP // 16, zbody, 0)

    ones16 = jnp.ones((16,), jnp.float32)

    def count_rows(n):
        def ebody(j, _):
            for cc in range(K // 16):
                idx = dst_v[j, pl.ds(cc * 16, 16)]
                plsc.addupdate_scatter(hist_v, [idx], ones16)
            return 0

        lax.fori_loop(0, n, ebody, 0)

    @pl.when(wid < NW - 1)
    def _():
        pltpu.sync_copy(ei_hbm.at[1, pl.ds(wid * NCHUNK, NCHUNK)], dst_v)
        count_rows(NCHUNK)

    @pl.when(wid == NW - 1)
    def _():
        pltpu.sync_copy(ei_hbm.at[1, pl.ds(31 * NCHUNK, R31)],
                        dst_v.at[pl.ds(0, R31)])
        count_rows(R31)
        pltpu.sync_copy(pad_dst_hbm, dst_v.at[pl.ds(0, PADC)])
        count_rows(PADC)

    pltpu.sync_copy(hist_v, out_hbm.at[wid])


@functools.partial(
    pl.kernel,
    out_type=jax.ShapeDtypeStruct((2, NP, C), jnp.float32),
    mesh=_mesh,
    scratch_types=[
        pltpu.VMEM((NCHUNK // 2, K), jnp.int32),
        pltpu.VMEM((NCHUNK // 2, K), jnp.int32),
        pltpu.VMEM((2, K, C), jnp.float32),
        pltpu.VMEM_SHARED((NP, C), jnp.float32),
        pltpu.SemaphoreType.DMA,
        pltpu.SemaphoreType.DMA,
    ],
)
def _sc_agg(hs_hbm, ei_hbm, pad_src_hbm, pad_dst_hbm, zeros_hbm, out_hbm,
            src_v, dst_v, rows_v, acc_sh, gsem, ssem):
    cid = lax.axis_index("c")
    sid = lax.axis_index("s")
    wid = sid * 2 + cid
    base = sid * ROWS_PT
    pltpu.sync_copy(zeros_hbm, acc_sh.at[pl.ds(base, ROWS_PT)])
    plsc.subcore_barrier()

    def start_g(j, b):
        pltpu.async_copy(hs_hbm.at[src_v.at[j]], rows_v.at[b], gsem)

    def wait_g(j, b):
        pltpu.make_async_copy(hs_hbm.at[src_v.at[j]], rows_v.at[b], gsem).wait()

    def start_s(j, b):
        pltpu.async_copy(rows_v.at[b], acc_sh.at[dst_v.at[j]], ssem, add=True)

    def wait_s(j, b):
        pltpu.make_async_copy(rows_v.at[b], acc_sh.at[dst_v.at[j]], ssem).wait()

    def run_phase(stage, n):
        stage()
        start_g(0, 0)
        wait_g(0, 0)
        start_s(0, 0)
        start_g(1, 1)

        def body(t, _):
            for i in range(2):
                j = 2 * t + 1 + i
                b = (1 + i) % 2
                wait_g(j, b)
                start_s(j, b)
                wait_s(j - 1, (b + 1) % 2)
                start_g(j + 1, (b + 1) % 2)
            return 0

        lax.fori_loop(0, (n - 2) // 2, body, 0)
        wait_g(n - 1, 1)
        start_s(n - 1, 1)
        wait_s(n - 2, 0)
        wait_s(n - 1, 1)

    def stage_real(start, n):
        def stage():
            pltpu.sync_copy(ei_hbm.at[0, pl.ds(start, n)],
                            src_v.at[pl.ds(0, n)])
            pltpu.sync_copy(ei_hbm.at[1, pl.ds(start, n)],
                            dst_v.at[pl.ds(0, n)])
        return stage

    def stage_pad(start, n):
        def stage():
            pltpu.sync_copy(pad_src_hbm.at[pl.ds(start, n)],
                            src_v.at[pl.ds(0, n)])
            pltpu.sync_copy(pad_dst_hbm.at[pl.ds(start, n)],
                            dst_v.at[pl.ds(0, n)])
        return stage

    @pl.when(wid < NW - 1)
    def _():
        s0 = wid * NCHUNK
        run_phase(stage_real(s0, 40), 40)
        run_phase(stage_real(s0 + 40, 40), 40)

    @pl.when(wid == NW - 1)
    def _():
        run_phase(stage_real(31 * NCHUNK, R31), R31)
        run_phase(stage_pad(0, 40), 40)
        run_phase(stage_pad(40, PADC - 40), PADC - 40)

    plsc.subcore_barrier()
    pltpu.sync_copy(acc_sh.at[pl.ds(base, ROWS_PT)],
                    out_hbm.at[cid, pl.ds(base, ROWS_PT)])


def _tc_matmul_body(x_ref, w_ref, h_ref):
    h_ref[...] = jnp.dot(x_ref[...], w_ref[...],
                         preferred_element_type=jnp.float32)


def _tc_matmul(x, W):
    return pl.pallas_call(
        _tc_matmul_body,
        grid=(N // BLK,),
        in_specs=[
            pl.BlockSpec((BLK, C), lambda i: (i, 0)),
            pl.BlockSpec((C, C), lambda i: (0, 0)),
        ],
        out_specs=pl.BlockSpec((BLK, C), lambda i: (i, 0)),
        out_shape=jax.ShapeDtypeStruct((N, C), jnp.float32),
    )(x, W)


def _tc_scale_body(h_ref, hists_ref, hs_ref, dfull_ref):
    deg = jnp.sum(hists_ref[...], axis=0) + 1.0
    d = lax.rsqrt(deg)[:N, None]
    hs_ref[...] = h_ref[...] * d
    dfull_ref[...] = jnp.broadcast_to(d, hs_ref.shape)


def _tc_scale(h, hists):
    return pl.pallas_call(
        _tc_scale_body,
        out_shape=[
            jax.ShapeDtypeStruct((N, C), jnp.float32),
            jax.ShapeDtypeStruct((N, C), jnp.float32),
        ],
    )(h, hists)


def _tc_final_body(acc0_ref, acc1_ref, hs_ref, df_ref, b_ref, pw_ref, o_ref):
    a0 = acc0_ref[0]
    a1 = acc1_ref[0]
    t = df_ref[...] * (a0 + a1 + hs_ref[...]) + b_ref[...]
    o_ref[...] = jnp.where(t >= 0, t, pw_ref[...] * t)


def _tc_final(acc, hs, dfull, b2, pw2):
    blk = pl.BlockSpec((BLK, C), lambda i: (i, 0))
    vec = pl.BlockSpec((1, C), lambda i: (0, 0))
    return pl.pallas_call(
        _tc_final_body,
        grid=(N // BLK,),
        in_specs=[
            pl.BlockSpec((1, BLK, C), lambda i: (0, i, 0)),
            pl.BlockSpec((1, BLK, C), lambda i: (1, i, 0)),
            blk, blk, vec, vec,
        ],
        out_specs=blk,
        out_shape=jax.ShapeDtypeStruct((N, C), jnp.float32),
    )(acc, acc, hs, dfull, b2, pw2)


def kernel(x, edge_index, W, b, prelu_w):
    ei3 = edge_index.astype(jnp.int32).reshape(2, RCHUNK, K)
    pad_iota = jnp.arange(PADC * K, dtype=jnp.int32)
    pad_src = (pad_iota % N).reshape(PADC, K)
    pad_dst = (N + pad_iota % (NP - N)).reshape(PADC, K)
    zeros_blk = jnp.zeros((ROWS_PT, C), jnp.float32)

    h = _tc_matmul(x, W)
    hists = _sc_hist(ei3, pad_dst)
    hs, dfull = _tc_scale(h, hists)
    acc = _sc_agg(hs, ei3, pad_src, pad_dst, zeros_blk)
    return _tc_final(acc, hs, dfull, b.reshape(1, C), prelu_w.reshape(1, C))


# speedup vs baseline: 47.7678x; 1.1435x over previous
import functools

import jax
import jax.numpy as jnp
from jax import lax
from jax.experimental import pallas as pl
from jax.experimental.pallas import tpu as pltpu
from jax.experimental.pallas import tpu_sc as plsc

N = 10000
E = 320000
C = 128
NP = 10240
NW = 32
K = 128
RCHUNK = E // K
NCHUNK = 80
PADC = NW * NCHUNK - RCHUNK
R31 = RCHUNK - 31 * NCHUNK
ROWS_PT = NP // 16
BLK = 1000

_mesh = plsc.VectorSubcoreMesh(core_axis_name="c", subcore_axis_name="s")


@functools.partial(
    pl.kernel,
    out_type=jax.ShapeDtypeStruct((NW, NP), jnp.float32),
    mesh=_mesh,
    scratch_types=[
        pltpu.VMEM((NCHUNK, K), jnp.int32),
        pltpu.VMEM((NP,), jnp.float32),
    ],
    compiler_params=pltpu.CompilerParams(needs_layout_passes=False),
)
def _sc_hist(ei_hbm, out_hbm, dst_v, hist_v):
    cid = lax.axis_index("c")
    sid = lax.axis_index("s")
    wid = sid * 2 + cid

    zero16 = jnp.zeros((16,), jnp.float32)

    def zbody(i, _):
        hist_v[pl.ds(i * 16, 16)] = zero16
        return 0

    lax.fori_loop(0, NP // 16, zbody, 0)

    ones16 = jnp.ones((16,), jnp.float32)

    def count_rows(n):
        def ebody(j, _):
            for cc in range(K // 16):
                idx = dst_v[j, pl.ds(cc * 16, 16)]
                plsc.addupdate_scatter(hist_v, [idx], ones16)
            return 0

        lax.fori_loop(0, n, ebody, 0)

    @pl.when(wid < NW - 1)
    def _():
        pltpu.sync_copy(ei_hbm.at[1, pl.ds(wid * NCHUNK, NCHUNK)], dst_v)
        count_rows(NCHUNK)

    @pl.when(wid == NW - 1)
    def _():
        pltpu.sync_copy(ei_hbm.at[1, pl.ds(31 * NCHUNK, R31)],
                        dst_v.at[pl.ds(0, R31)])
        count_rows(R31)

    pltpu.sync_copy(hist_v, out_hbm.at[wid])


@functools.partial(
    pl.kernel,
    out_type=jax.ShapeDtypeStruct((2, NP, C), jnp.float32),
    mesh=_mesh,
    scratch_types=[
        pltpu.VMEM((NCHUNK // 2, K), jnp.int32),
        pltpu.VMEM((NCHUNK // 2, K), jnp.int32),
        pltpu.VMEM((2, K, C), jnp.float32),
        pltpu.VMEM_SHARED((NP, C), jnp.float32),
        pltpu.SemaphoreType.DMA,
        pltpu.SemaphoreType.DMA,
    ],
)
def _sc_agg(hs_hbm, ei_hbm, zeros_hbm, out_hbm,
            src_v, dst_v, rows_v, acc_sh, gsem, ssem):
    cid = lax.axis_index("c")
    sid = lax.axis_index("s")
    wid = sid * 2 + cid
    base = sid * ROWS_PT
    pltpu.sync_copy(zeros_hbm, acc_sh.at[pl.ds(base, ROWS_PT)])
    plsc.subcore_barrier()

    def start_g(j, b):
        pltpu.async_copy(hs_hbm.at[src_v.at[j]], rows_v.at[b], gsem)

    def wait_g(j, b):
        pltpu.make_async_copy(hs_hbm.at[src_v.at[j]], rows_v.at[b], gsem).wait()

    def start_s(j, b):
        pltpu.async_copy(rows_v.at[b], acc_sh.at[dst_v.at[j]], ssem, add=True)

    def wait_s(j, b):
        pltpu.make_async_copy(rows_v.at[b], acc_sh.at[dst_v.at[j]], ssem).wait()

    def run_phase(start, n):
        pltpu.sync_copy(ei_hbm.at[0, pl.ds(start, n)],
                        src_v.at[pl.ds(0, n)])
        pltpu.sync_copy(ei_hbm.at[1, pl.ds(start, n)],
                        dst_v.at[pl.ds(0, n)])
        start_g(0, 0)
        start_g(1, 1)
        wait_g(0, 0)
        start_s(0, 0)

        def body(t, _):
            for i in range(2):
                j = 2 * t + 1 + i
                b = (1 + i) % 2
                wait_s(j - 1, (b + 1) % 2)
                start_g(j + 1, (b + 1) % 2)
                wait_g(j, b)
                start_s(j, b)
            return 0

        lax.fori_loop(0, (n - 2) // 2, body, 0)
        wait_g(n - 1, 1)
        start_s(n - 1, 1)
        wait_s(n - 2, 0)
        wait_s(n - 1, 1)

    @pl.when(wid < NW - 1)
    def _():
        s0 = wid * NCHUNK
        run_phase(s0, 40)
        run_phase(s0 + 40, 40)

    @pl.when(wid == NW - 1)
    def _():
        run_phase(31 * NCHUNK, R31)

    plsc.subcore_barrier()
    pltpu.sync_copy(acc_sh.at[pl.ds(base, ROWS_PT)],
                    out_hbm.at[cid, pl.ds(base, ROWS_PT)])


def _tc_matmul_body(x_ref, w_ref, h_ref):
    h_ref[...] = jnp.dot(x_ref[...], w_ref[...],
                         preferred_element_type=jnp.float32)


def _tc_matmul(x, W):
    return pl.pallas_call(
        _tc_matmul_body,
        grid=(N // BLK,),
        in_specs=[
            pl.BlockSpec((BLK, C), lambda i: (i, 0)),
            pl.BlockSpec((C, C), lambda i: (0, 0)),
        ],
        out_specs=pl.BlockSpec((BLK, C), lambda i: (i, 0)),
        out_shape=jax.ShapeDtypeStruct((N, C), jnp.float32),
    )(x, W)


def _tc_scale_body(h_ref, hists_ref, hs_ref, dfull_ref):
    deg = jnp.sum(hists_ref[...], axis=0) + 1.0
    d = lax.rsqrt(deg)[:N, None]
    hs_ref[...] = h_ref[...] * d
    dfull_ref[...] = jnp.broadcast_to(d, hs_ref.shape)


def _tc_scale(h, hists):
    return pl.pallas_call(
        _tc_scale_body,
        out_shape=[
            jax.ShapeDtypeStruct((N, C), jnp.float32),
            jax.ShapeDtypeStruct((N, C), jnp.float32),
        ],
    )(h, hists)


def _tc_final_body(acc0_ref, acc1_ref, hs_ref, df_ref, b_ref, pw_ref, o_ref):
    a0 = acc0_ref[0]
    a1 = acc1_ref[0]
    t = df_ref[...] * (a0 + a1 + hs_ref[...]) + b_ref[...]
    o_ref[...] = jnp.where(t >= 0, t, pw_ref[...] * t)


def _tc_final(acc, hs, dfull, b2, pw2):
    blk = pl.BlockSpec((BLK, C), lambda i: (i, 0))
    vec = pl.BlockSpec((1, C), lambda i: (0, 0))
    return pl.pallas_call(
        _tc_final_body,
        grid=(N // BLK,),
        in_specs=[
            pl.BlockSpec((1, BLK, C), lambda i: (0, i, 0)),
            pl.BlockSpec((1, BLK, C), lambda i: (1, i, 0)),
            blk, blk, vec, vec,
        ],
        out_specs=blk,
        out_shape=jax.ShapeDtypeStruct((N, C), jnp.float32),
    )(acc, acc, hs, dfull, b2, pw2)


def kernel(x, edge_index, W, b, prelu_w):
    ei3 = edge_index.astype(jnp.int32).reshape(2, RCHUNK, K)
    zeros_blk = jnp.zeros((ROWS_PT, C), jnp.float32)

    h = _tc_matmul(x, W)
    hists = _sc_hist(ei3)
    hs, dfull = _tc_scale(h, hists)
    acc = _sc_agg(hs, ei3, zeros_blk)
    return _tc_final(acc, hs, dfull, b.reshape(1, C), prelu_w.reshape(1, C))
